# baseline (speedup 1.0000x reference)
import functools

import jax
import jax.numpy as jnp
import numpy as np
from jax import lax
from jax.experimental import pallas as pl
from jax.experimental.pallas import tpu as pltpu
from jax.experimental.pallas import tpu_sc as plsc

_F32 = jnp.float32
_I32 = jnp.int32

D = 64
B = 16384
V = 100000
NC, NS = 2, 16
NW = NC * NS

TBLK = 128
NBLK = 781
VP = (NBLK + 1) * TBLK
BPW = 26

ROWS_PER_W = B // NW
CHUNK = 128
NCHUNK = ROWS_PER_W // CHUNK
GROUPS = CHUNK // 16
UNROLL = 4

_EPS2 = np.float32(1e-24)
_TINY = np.float32(1e-30)


def _rsqrt(x):
    i = plsc.bitcast(x, _I32)
    i = np.int32(0x5F3759DF) - (i >> 1)
    y = plsc.bitcast(i, _F32)
    for _ in range(3):
        y = y * (np.float32(1.5) - np.float32(0.5) * x * y * y)
    return y


def _transpose_block(in_v, out_v):
    lanes = lax.iota(_I32, 16)
    tun = 8

    def grp(g, carry):
        cols = g * 16 + lanes
        cols64 = cols * 64

        def dim(j, carry2):
            base = lanes + j * tun
            vals = []
            for u in range(tun):
                dims = (base + u) & 63
                vals.append((dims, plsc.load_gather(in_v, [dims, cols])))
            for dims, v in vals:
                plsc.store_scatter(out_v, [cols64 + dims], v)
            return carry2

        lax.fori_loop(0, D // tun, dim, 0)
        return carry

    lax.fori_loop(0, TBLK // 16, grp, 0)


NIB = 3


def _t_body(eT, rT, nT, e_out, r_out, n_out,
            in_v0, in_v1, in_v2, out_v0, out_v1, isems, osems):
    wid = lax.axis_index("s") * NC + lax.axis_index("c")
    in_bufs = (in_v0, in_v1, in_v2)
    out_bufs = (out_v0, out_v1)
    tables = ((eT, e_out), (rT, r_out), (nT, n_out))
    slots = [(tables[s // BPW], s % BPW) for s in range(3 * BPW)]

    def blk(i):
        return jnp.minimum(i * NW + wid, NBLK)

    def start_in(src, i, b):
        return pltpu.async_copy(src.at[:, pl.ds(blk(i) * TBLK, TBLK)],
                                in_bufs[b], isems.at[b])

    in_cps = [None] * NIB
    for p in range(NIB - 1):
        (psrc, _), pi = slots[p]
        in_cps[p] = start_in(psrc, pi, p)
    out_cps = [None, None]
    for s, ((src, dst), i) in enumerate(slots):
        b = s % NIB
        if s + NIB - 1 < len(slots):
            (nsrc, _), ni = slots[s + NIB - 1]
            nb = (s + NIB - 1) % NIB
            in_cps[nb] = start_in(nsrc, ni, nb)
        in_cps[b].wait()
        ob = s % 2
        if out_cps[ob] is not None:
            out_cps[ob].wait()
        _transpose_block(in_bufs[b], out_bufs[ob])
        out_cps[ob] = pltpu.async_copy(
            out_bufs[ob], dst.at[pl.ds(blk(i) * (TBLK * D), TBLK * D)],
            osems.at[ob])
    for cp in out_cps:
        cp.wait()


_transpose = functools.partial(
    pl.kernel,
    mesh=plsc.VectorSubcoreMesh(core_axis_name="c", subcore_axis_name="s"),
    out_type=(jax.ShapeDtypeStruct((VP * D,), _F32),) * 3,
    compiler_params=pltpu.CompilerParams(
        needs_layout_passes=False, use_tc_tiling_on_sc=True,
        disable_bounds_checks=True),
    scratch_types=[
        pltpu.VMEM((D, TBLK), _F32),
        pltpu.VMEM((D, TBLK), _F32),
        pltpu.VMEM((D, TBLK), _F32),
        pltpu.VMEM((TBLK * D,), _F32),
        pltpu.VMEM((TBLK * D,), _F32),
        pltpu.SemaphoreType.DMA((3,)),
        pltpu.SemaphoreType.DMA((2,)),
    ],
)(_t_body)


def _extract_indices(slab_v, ih_v, ir_v, it_v):
    lanes = lax.iota(_I32, 16)

    def body(g, carry):
        rows = g * 16 + lanes
        for col, dst in ((0, ih_v), (1, ir_v), (2, it_v)):
            c = jnp.zeros((16,), _I32) + col
            dst[pl.ds(g * 16, 16)] = plsc.load_gather(slab_v, [rows, c])
        return carry

    lax.fori_loop(0, GROUPS, body, 0)


def _compute_chunk(c, h_v, r_v, t_v, w_v, s_v):
    lanes = lax.iota(_I32, 16)
    zero = jnp.zeros((16,), _F32)

    def group_body(g, carry):
        rows = g * 16 + lanes

        def dim_body(j, acc):
            hh, tt, rr, ww, hw, tw, hr, tr, ht, wr = acc
            for u in range(UNROLL):
                col = (lanes + (j * UNROLL + u)) & 63
                hd = plsc.load_gather(h_v, [rows, col])
                rd = plsc.load_gather(r_v, [rows, col])
                td = plsc.load_gather(t_v, [rows, col])
                wd = plsc.load_gather(w_v, [rows, col])
                hh = hh + hd * hd
                tt = tt + td * td
                rr = rr + rd * rd
                ww = ww + wd * wd
                hw = hw + hd * wd
                tw = tw + td * wd
                hr = hr + hd * rd
                tr = tr + td * rd
                ht = ht + hd * td
                wr = wr + wd * rd
            return (hh, tt, rr, ww, hw, tw, hr, tr, ht, wr)

        hh, tt, rr, ww, hw, tw, hr, tr, ht, wr = lax.fori_loop(
            0, D // UNROLL, dim_body, (zero,) * 10)

        s = _rsqrt(jnp.maximum(ww, _EPS2))
        a = hw * s
        b = tw * s
        p2 = jnp.maximum(hh - a * a, np.float32(0.0))
        q2 = jnp.maximum(tt - b * b, np.float32(0.0))
        p = _rsqrt(jnp.maximum(p2, _EPS2))
        q = _rsqrt(jnp.maximum(q2, _EPS2))
        hvr = hr - a * s * wr
        tvr = tr - b * s * wr
        hvtv = ht - a * b
        d2 = (p2 * p * p + rr + q2 * q * q
              + np.float32(2.0) * (p * hvr - p * q * hvtv - q * tvr))
        d2 = jnp.maximum(d2, np.float32(0.0))
        score = d2 * _rsqrt(jnp.maximum(d2, _TINY))
        s_v[pl.ds(c * CHUNK + g * 16, 16)] = score
        return carry

    lax.fori_loop(0, GROUPS, group_body, 0)


def _body(sample, ent, rel, nv, out,
          slab_v, ih_v, ir_v, it_v, h_v, r_v, t_v, w_v, s_v, sems):
    wid = lax.axis_index("s") * NC + lax.axis_index("c")
    base = wid * ROWS_PER_W

    def stage(c):
        d = c % 2
        pltpu.sync_copy(sample.at[pl.ds(base + c * CHUNK, CHUNK), :],
                        slab_v.at[d])
        _extract_indices(slab_v.at[d], ih_v.at[d], ir_v.at[d], it_v.at[d])
        return [pltpu.async_copy(ent.at[ih_v.at[d]], h_v.at[d], sems.at[d]),
                pltpu.async_copy(rel.at[ir_v.at[d]], r_v.at[d], sems.at[d]),
                pltpu.async_copy(nv.at[ir_v.at[d]], w_v.at[d], sems.at[d]),
                pltpu.async_copy(ent.at[it_v.at[d]], t_v.at[d], sems.at[d])]

    inflight = stage(0)
    for c in range(NCHUNK):
        nxt = stage(c + 1) if c + 1 < NCHUNK else None
        for cp in inflight:
            cp.wait()
        d = c % 2
        _compute_chunk(c, h_v.at[d], r_v.at[d], t_v.at[d], w_v.at[d], s_v)
        inflight = nxt
    pltpu.sync_copy(s_v, out.at[pl.ds(base, ROWS_PER_W)])


_transh = functools.partial(
    pl.kernel,
    mesh=plsc.VectorSubcoreMesh(core_axis_name="c", subcore_axis_name="s"),
    out_type=jax.ShapeDtypeStruct((B,), _F32),
    compiler_params=pltpu.CompilerParams(
        needs_layout_passes=False, use_tc_tiling_on_sc=False),
    scratch_types=[
        pltpu.VMEM((2, CHUNK, 3), _I32),
        pltpu.VMEM((2, CHUNK), _I32),
        pltpu.VMEM((2, CHUNK), _I32),
        pltpu.VMEM((2, CHUNK), _I32),
        pltpu.VMEM((2, CHUNK, D), _F32),
        pltpu.VMEM((2, CHUNK, D), _F32),
        pltpu.VMEM((2, CHUNK, D), _F32),
        pltpu.VMEM((2, CHUNK, D), _F32),
        pltpu.VMEM((ROWS_PER_W,), _F32),
        pltpu.SemaphoreType.DMA((2,)),
    ],
)(_body)


def kernel(sample, entity_embedding, translation_embedding, norm_vector):
    e_lin, r_lin, n_lin = _transpose(jnp.transpose(entity_embedding),
                                     jnp.transpose(translation_embedding),
                                     jnp.transpose(norm_vector))
    return _transh(sample.astype(_I32),
                   e_lin.reshape(VP, D),
                   r_lin.reshape(VP, D),
                   n_lin.reshape(VP, D))


# speedup vs baseline: 1.1207x; 1.1207x over previous
import functools

import jax
import jax.numpy as jnp
import numpy as np
from jax import lax
from jax.experimental import pallas as pl
from jax.experimental.pallas import tpu as pltpu
from jax.experimental.pallas import tpu_sc as plsc

_F32 = jnp.float32
_I32 = jnp.int32

D = 64
B = 16384
V = 100000
NC, NS = 2, 16
NW = NC * NS

TBLK = 128
NBLK = 781
VP = (NBLK + 1) * TBLK
BPW = 26

ROWS_PER_W = B // NW
CHUNK = 128
NCHUNK = ROWS_PER_W // CHUNK
GROUPS = CHUNK // 16
UNROLL = 4

_EPS2 = np.float32(1e-24)
_TINY = np.float32(1e-30)


def _rsqrt(x):
    i = plsc.bitcast(x, _I32)
    i = np.int32(0x5F3759DF) - (i >> 1)
    y = plsc.bitcast(i, _F32)
    for _ in range(3):
        y = y * (np.float32(1.5) - np.float32(0.5) * x * y * y)
    return y


def _transpose_block(in_v, out_v):
    lanes = lax.iota(_I32, 16)
    tun = 8

    def grp(g, carry):
        cols = g * 16 + lanes
        cols64 = cols * 64

        def dim(j, carry2):
            base = lanes + j * tun
            vals = []
            for u in range(tun):
                dims = (base + u) & 63
                vals.append((dims, plsc.load_gather(in_v, [dims, cols])))
            for dims, v in vals:
                plsc.store_scatter(out_v, [cols64 + dims], v)
            return carry2

        lax.fori_loop(0, D // tun, dim, 0)
        return carry

    lax.fori_loop(0, TBLK // 16, grp, 0)


NIB = 3


def _t_body(eT, rT, nT, e_out, r_out, n_out,
            in_v0, in_v1, in_v2, out_v0, out_v1, isems, osems):
    wid = lax.axis_index("s") * NC + lax.axis_index("c")
    in_bufs = (in_v0, in_v1, in_v2)
    out_bufs = (out_v0, out_v1)
    tables = ((eT, e_out), (rT, r_out), (nT, n_out))
    slots = [(tables[s // BPW], s % BPW) for s in range(3 * BPW)]

    def blk(i):
        return jnp.minimum(wid * BPW + i, NBLK)

    def start_in(src, i, b):
        return pltpu.async_copy(src.at[:, pl.ds(blk(i) * TBLK, TBLK)],
                                in_bufs[b], isems.at[b])

    in_cps = [None] * NIB
    for p in range(NIB - 1):
        (psrc, _), pi = slots[p]
        in_cps[p] = start_in(psrc, pi, p)
    out_cps = [None, None]
    for s, ((src, dst), i) in enumerate(slots):
        b = s % NIB
        if s + NIB - 1 < len(slots):
            (nsrc, _), ni = slots[s + NIB - 1]
            nb = (s + NIB - 1) % NIB
            in_cps[nb] = start_in(nsrc, ni, nb)
        in_cps[b].wait()
        ob = s % 2
        if out_cps[ob] is not None:
            out_cps[ob].wait()
        _transpose_block(in_bufs[b], out_bufs[ob])
        out_cps[ob] = pltpu.async_copy(
            out_bufs[ob], dst.at[pl.ds(blk(i) * (TBLK * D), TBLK * D)],
            osems.at[ob])
    for cp in out_cps:
        cp.wait()


_transpose = functools.partial(
    pl.kernel,
    mesh=plsc.VectorSubcoreMesh(core_axis_name="c", subcore_axis_name="s"),
    out_type=(jax.ShapeDtypeStruct((VP * D,), _F32),) * 3,
    compiler_params=pltpu.CompilerParams(
        needs_layout_passes=False, use_tc_tiling_on_sc=True,
        disable_bounds_checks=True),
    scratch_types=[
        pltpu.VMEM((D, TBLK), _F32),
        pltpu.VMEM((D, TBLK), _F32),
        pltpu.VMEM((D, TBLK), _F32),
        pltpu.VMEM((TBLK * D,), _F32),
        pltpu.VMEM((TBLK * D,), _F32),
        pltpu.SemaphoreType.DMA((3,)),
        pltpu.SemaphoreType.DMA((2,)),
    ],
)(_t_body)


def _extract_indices(slab_v, ih_v, ir_v, it_v):
    lanes = lax.iota(_I32, 16)

    def body(g, carry):
        rows = g * 16 + lanes
        for col, dst in ((0, ih_v), (1, ir_v), (2, it_v)):
            c = jnp.zeros((16,), _I32) + col
            dst[pl.ds(g * 16, 16)] = plsc.load_gather(slab_v, [rows, c])
        return carry

    lax.fori_loop(0, GROUPS, body, 0)


def _compute_chunk(c, h_v, r_v, t_v, w_v, s_v):
    lanes = lax.iota(_I32, 16)
    zero = jnp.zeros((16,), _F32)

    def group_body(g, carry):
        rows = g * 16 + lanes

        def dim_body(j, acc):
            hh, tt, rr, ww, hw, tw, hr, tr, ht, wr = acc
            for u in range(UNROLL):
                col = (lanes + (j * UNROLL + u)) & 63
                hd = plsc.load_gather(h_v, [rows, col])
                rd = plsc.load_gather(r_v, [rows, col])
                td = plsc.load_gather(t_v, [rows, col])
                wd = plsc.load_gather(w_v, [rows, col])
                hh = hh + hd * hd
                tt = tt + td * td
                rr = rr + rd * rd
                ww = ww + wd * wd
                hw = hw + hd * wd
                tw = tw + td * wd
                hr = hr + hd * rd
                tr = tr + td * rd
                ht = ht + hd * td
                wr = wr + wd * rd
            return (hh, tt, rr, ww, hw, tw, hr, tr, ht, wr)

        hh, tt, rr, ww, hw, tw, hr, tr, ht, wr = lax.fori_loop(
            0, D // UNROLL, dim_body, (zero,) * 10)

        s = _rsqrt(jnp.maximum(ww, _EPS2))
        a = hw * s
        b = tw * s
        p2 = jnp.maximum(hh - a * a, np.float32(0.0))
        q2 = jnp.maximum(tt - b * b, np.float32(0.0))
        p = _rsqrt(jnp.maximum(p2, _EPS2))
        q = _rsqrt(jnp.maximum(q2, _EPS2))
        hvr = hr - a * s * wr
        tvr = tr - b * s * wr
        hvtv = ht - a * b
        d2 = (p2 * p * p + rr + q2 * q * q
              + np.float32(2.0) * (p * hvr - p * q * hvtv - q * tvr))
        d2 = jnp.maximum(d2, np.float32(0.0))
        score = d2 * _rsqrt(jnp.maximum(d2, _TINY))
        s_v[pl.ds(c * CHUNK + g * 16, 16)] = score
        return carry

    lax.fori_loop(0, GROUPS, group_body, 0)


def _body(sample, ent, rel, nv, out,
          slab_v, ih_v, ir_v, it_v, h_v, r_v, t_v, w_v, s_v, sems):
    wid = lax.axis_index("s") * NC + lax.axis_index("c")
    base = wid * ROWS_PER_W

    def stage(c):
        d = c % 2
        pltpu.sync_copy(sample.at[pl.ds(base + c * CHUNK, CHUNK), :],
                        slab_v.at[d])
        _extract_indices(slab_v.at[d], ih_v.at[d], ir_v.at[d], it_v.at[d])
        return [pltpu.async_copy(ent.at[ih_v.at[d]], h_v.at[d], sems.at[d]),
                pltpu.async_copy(rel.at[ir_v.at[d]], r_v.at[d], sems.at[d]),
                pltpu.async_copy(nv.at[ir_v.at[d]], w_v.at[d], sems.at[d]),
                pltpu.async_copy(ent.at[it_v.at[d]], t_v.at[d], sems.at[d])]

    inflight = stage(0)
    for c in range(NCHUNK):
        nxt = stage(c + 1) if c + 1 < NCHUNK else None
        for cp in inflight:
            cp.wait()
        d = c % 2
        _compute_chunk(c, h_v.at[d], r_v.at[d], t_v.at[d], w_v.at[d], s_v)
        inflight = nxt
    pltpu.sync_copy(s_v, out.at[pl.ds(base, ROWS_PER_W)])


_transh = functools.partial(
    pl.kernel,
    mesh=plsc.VectorSubcoreMesh(core_axis_name="c", subcore_axis_name="s"),
    out_type=jax.ShapeDtypeStruct((B,), _F32),
    compiler_params=pltpu.CompilerParams(
        needs_layout_passes=False, use_tc_tiling_on_sc=False),
    scratch_types=[
        pltpu.VMEM((2, CHUNK, 3), _I32),
        pltpu.VMEM((2, CHUNK), _I32),
        pltpu.VMEM((2, CHUNK), _I32),
        pltpu.VMEM((2, CHUNK), _I32),
        pltpu.VMEM((2, CHUNK, D), _F32),
        pltpu.VMEM((2, CHUNK, D), _F32),
        pltpu.VMEM((2, CHUNK, D), _F32),
        pltpu.VMEM((2, CHUNK, D), _F32),
        pltpu.VMEM((ROWS_PER_W,), _F32),
        pltpu.SemaphoreType.DMA((2,)),
    ],
)(_body)


def kernel(sample, entity_embedding, translation_embedding, norm_vector):
    e_lin, r_lin, n_lin = _transpose(jnp.transpose(entity_embedding),
                                     jnp.transpose(translation_embedding),
                                     jnp.transpose(norm_vector))
    return _transh(sample.astype(_I32),
                   e_lin.reshape(VP, D),
                   r_lin.reshape(VP, D),
                   n_lin.reshape(VP, D))
